# baseline (speedup 1.0000x reference)
import functools

import jax
import jax.numpy as jnp
from jax import lax
from jax.experimental import pallas as pl
from jax.experimental.pallas import tpu as pltpu
from jax.experimental.pallas import tpu_sc as plsc

BZ = 4
H = 128
W = 128
V = 100000
C = 64
P = H * W
NW = 32
POS_PER_W = BZ * P // NW
CHUNK = 512
N_CHUNKS = POS_PER_W // CHUNK
IDX_MINOR = 128


def _body(kg_hbm, idx_hbm, out_hbm, idx_v, rows_v, rowsT_v, sem):
    wid = lax.axis_index("s") * 2 + lax.axis_index("c")
    b = wid // (NW // BZ)
    p_base = (wid % (NW // BZ)) * POS_PER_W
    cidx = [lax.iota(jnp.int32, 16) + 16 * g for g in range(C // 16)]

    pltpu.sync_copy(
        idx_hbm.at[pl.ds(wid * (POS_PER_W // IDX_MINOR), POS_PER_W // IDX_MINOR)],
        idx_v,
    )

    for k in range(N_CHUNKS):
        p0 = p_base + k * CHUNK
        copies = [
            pltpu.async_copy(
                kg_hbm.at[idx_v.at[k * (CHUNK // IDX_MINOR) + j]],
                rows_v.at[pl.ds(j * IDX_MINOR, IDX_MINOR)],
                sem,
            )
            for j in range(CHUNK // IDX_MINOR)
        ]
        for cp in copies:
            cp.wait()

        def tr(i, _):
            pvec = jnp.full((16,), i, dtype=jnp.int32)
            for g in range(C // 16):
                vec = rows_v[i, pl.ds(g * 16, 16)]
                plsc.store_scatter(rowsT_v, [cidx[g], pvec], vec)
            return 0

        lax.fori_loop(0, CHUNK, tr, 0, unroll=4)

        pltpu.sync_copy(rowsT_v, out_hbm.at[b, :, pl.ds(p0, CHUNK)])


@jax.jit
def kernel(kg_node_feats, obs):
    kg_flat = kg_node_feats.reshape(BZ * V, C)
    idx = obs.astype(jnp.int32).reshape(BZ, P)
    idx = idx + (jnp.arange(BZ, dtype=jnp.int32) * V)[:, None]
    idx = idx.reshape(BZ * P // IDX_MINOR, IDX_MINOR)

    mesh = plsc.VectorSubcoreMesh(core_axis_name="c", subcore_axis_name="s")
    out = pl.kernel(
        _body,
        out_type=jax.ShapeDtypeStruct((BZ, C, P), jnp.float32),
        mesh=mesh,
        compiler_params=pltpu.CompilerParams(
            use_tc_tiling_on_sc=False, needs_layout_passes=False
        ),
        scratch_types=[
            pltpu.VMEM((POS_PER_W // IDX_MINOR, IDX_MINOR), jnp.int32),
            pltpu.VMEM((CHUNK, C), jnp.float32),
            pltpu.VMEM((C, CHUNK), jnp.float32),
            pltpu.SemaphoreType.DMA,
        ],
    )(kg_flat, idx)
    return out.reshape(BZ, C, H, W)


# speedup vs baseline: 3.9977x; 3.9977x over previous
import jax
import jax.numpy as jnp
from jax import lax
from jax.experimental import pallas as pl
from jax.experimental.pallas import tpu as pltpu
from jax.experimental.pallas import tpu_sc as plsc

BZ, V, C, P = 4, 100000, 64, 16384
H = W = 128
NW = 32
ROWS_PER_W = BZ * C // NW
OUT_CH = 8192
N_CH = P // OUT_CH
OUT_ROWS = OUT_CH // W


def _body(kg_hbm, idx_hbm, out_hbm, idx_v, row_v, out_v, sem):
    wid = lax.axis_index("s") * 2 + lax.axis_index("c")
    b = wid // (NW // BZ)
    pltpu.sync_copy(idx_hbm.at[b], idx_v)

    for j in range(ROWS_PER_W):
        r = wid * ROWS_PER_W + j
        pltpu.async_copy(kg_hbm.at[r], row_v, sem).wait()
        for h in range(N_CH):
            @plsc.parallel_loop(0, OUT_ROWS, unroll=4)
            def gath(w):
                base = h * OUT_CH + w * W
                for jj in range(W // 16):
                    vec = plsc.load_gather(
                        row_v, [idx_v[pl.ds(base + jj * 16, 16)]]
                    )
                    out_v[w, pl.ds(jj * 16, 16)] = vec
            pltpu.sync_copy(
                out_v, out_hbm.at[pl.ds(r * H + h * OUT_ROWS, OUT_ROWS), :]
            )


@jax.jit
def kernel(kg_node_feats, obs):
    kg_cm = jnp.transpose(kg_node_feats, (0, 2, 1)).reshape(BZ * C, V)
    idx = obs.astype(jnp.int32).reshape(BZ, P)

    mesh = plsc.VectorSubcoreMesh(core_axis_name="c", subcore_axis_name="s")
    out = pl.kernel(
        _body,
        out_type=jax.ShapeDtypeStruct((BZ * C * H, W), jnp.float32),
        mesh=mesh,
        compiler_params=pltpu.CompilerParams(
            use_tc_tiling_on_sc=True, needs_layout_passes=False
        ),
        scratch_types=[
            pltpu.VMEM((P,), jnp.int32),
            pltpu.VMEM((V,), jnp.float32),
            pltpu.VMEM((OUT_ROWS, W), jnp.float32),
            pltpu.SemaphoreType.DMA,
        ],
    )(kg_cm, idx)
    return out.reshape(BZ, C, H, W)


# speedup vs baseline: 4.0003x; 1.0007x over previous
import jax
import jax.numpy as jnp
from jax import lax
from jax.experimental import pallas as pl
from jax.experimental.pallas import tpu as pltpu
from jax.experimental.pallas import tpu_sc as plsc

BZ, V, C, P = 4, 100000, 64, 16384
H = W = 128
NW = 32
ROWS_PER_W = BZ * C // NW
OUT_CH = 8192
N_CH = P // OUT_CH
OUT_ROWS = OUT_CH // W


def _body(kg_hbm, idx_hbm, out_hbm, idx_v, row_v, out_v, sem, osem):
    wid = lax.axis_index("s") * 2 + lax.axis_index("c")
    b = wid // (NW // BZ)
    pltpu.sync_copy(idx_hbm.at[b], idx_v)

    out_dma = [None]
    for j in range(ROWS_PER_W):
        r = wid * ROWS_PER_W + j
        pltpu.async_copy(kg_hbm.at[r], row_v, sem).wait()
        for h in range(N_CH):
            if out_dma[0] is not None:
                out_dma[0].wait()

            @plsc.parallel_loop(0, OUT_ROWS, unroll=8)
            def gath(w):
                for jj in range(W // 16):
                    vec = plsc.load_gather(
                        row_v, [idx_v[h * OUT_ROWS + w, pl.ds(jj * 16, 16)]]
                    )
                    out_v[w, pl.ds(jj * 16, 16)] = vec

            out_dma[0] = pltpu.async_copy(
                out_v,
                out_hbm.at[pl.ds(r * H + h * OUT_ROWS, OUT_ROWS), :],
                osem,
            )
    out_dma[0].wait()


@jax.jit
def kernel(kg_node_feats, obs):
    kg_cm = jnp.transpose(kg_node_feats, (0, 2, 1)).reshape(BZ * C, V)
    idx = obs.astype(jnp.int32)

    mesh = plsc.VectorSubcoreMesh(core_axis_name="c", subcore_axis_name="s")
    out = pl.kernel(
        _body,
        out_type=jax.ShapeDtypeStruct((BZ * C * H, W), jnp.float32),
        mesh=mesh,
        compiler_params=pltpu.CompilerParams(
            use_tc_tiling_on_sc=True, needs_layout_passes=False
        ),
        scratch_types=[
            pltpu.VMEM((H, W), jnp.int32),
            pltpu.VMEM((V,), jnp.float32),
            pltpu.VMEM((OUT_ROWS, W), jnp.float32),
            pltpu.SemaphoreType.DMA,
            pltpu.SemaphoreType.DMA,
        ],
    )(kg_cm, idx)
    return out.reshape(BZ, C, H, W)
